# Initial kernel scaffold; baseline (speedup 1.0000x reference)
import jax
import jax.numpy as jnp
from jax.experimental import pallas as pl


def kernel(feat, segment_ids, W_gate, W_feat, b_feat):
    raise NotImplementedError("write your pallas kernel here")


# speedup vs baseline: 6.6726x; 6.6726x over previous
import jax
import jax.numpy as jnp
from jax.experimental import pallas as pl
from jax.experimental.pallas import tpu as pltpu

_N = 100000
_D = 128
_H = 128
_B = 64
_R = 2000
_NBLK = _N // _R

_HI = jax.lax.Precision.HIGHEST


def _body(ids_ref, feat_ref, wg_ref, wf_ref, bf_ref, out_ref,
          m_ref, s_ref, v_ref):
    i = pl.program_id(0)
    nb = pl.num_programs(0)

    @pl.when(i == 0)
    def _init():
        m_ref[...] = jnp.full_like(m_ref, -jnp.inf)
        s_ref[...] = jnp.zeros_like(s_ref)
        v_ref[...] = jnp.zeros_like(v_ref)

    feat = feat_ref[...]
    ids = ids_ref[0, 0, :]

    g = jax.lax.dot_general(wg_ref[...], feat, (((0,), (1,)), ((), ())),
                            preferred_element_type=jnp.float32,
                            precision=_HI)

    onehot_b = jax.lax.broadcasted_iota(jnp.int32, (_B, _R), 0) == ids[None, :]
    onehot = onehot_b.astype(jnp.float32)

    m_old = m_ref[...]
    m_blk = jnp.max(jnp.where(onehot_b, g, -jnp.inf), axis=1, keepdims=True)
    m_new = jnp.maximum(m_old, m_blk)
    scale = jnp.where(m_old == -jnp.inf, 0.0, jnp.exp(m_old - m_new))

    m_safe = jnp.where(m_new == -jnp.inf, 0.0, m_new)
    m_gat = jax.lax.dot_general(m_safe, onehot, (((0,), (0,)), ((), ())),
                                preferred_element_type=jnp.float32,
                                precision=_HI)
    e = jnp.exp(g - m_gat)
    we = onehot * e

    s_ref[...] = s_ref[...] * scale + jnp.sum(we, axis=1, keepdims=True)
    v_blk = jax.lax.dot_general(we, feat, (((1,), (0,)), ((), ())),
                                preferred_element_type=jnp.float32,
                                precision=_HI)
    v_ref[...] = v_ref[...] * scale + v_blk
    m_ref[...] = m_new

    @pl.when(i == nb - 1)
    def _finish():
        s = s_ref[...]
        pooled = jnp.where(s > 0, v_ref[...] / jnp.where(s > 0, s, 1.0), 0.0)
        out_ref[...] = jax.lax.dot_general(
            pooled, wf_ref[...], (((1,), (0,)), ((), ())),
            preferred_element_type=jnp.float32, precision=_HI) + bf_ref[...]


def kernel(feat, segment_ids, W_gate, W_feat, b_feat):
    ids3 = segment_ids.reshape(_NBLK, 1, _R)
    bf2 = b_feat.reshape(1, _H)
    return pl.pallas_call(
        _body,
        grid=(_NBLK,),
        in_specs=[
            pl.BlockSpec((1, 1, _R), lambda i: (i, 0, 0)),
            pl.BlockSpec((_R, _D), lambda i: (i, 0)),
            pl.BlockSpec((_D, 1), lambda i: (0, 0)),
            pl.BlockSpec((_D, _H), lambda i: (0, 0)),
            pl.BlockSpec((1, _H), lambda i: (0, 0)),
        ],
        out_specs=pl.BlockSpec((_B, _H), lambda i: (0, 0)),
        out_shape=jax.ShapeDtypeStruct((_B, _H), jnp.float32),
        scratch_shapes=[
            pltpu.VMEM((_B, 1), jnp.float32),
            pltpu.VMEM((_B, 1), jnp.float32),
            pltpu.VMEM((_B, _H), jnp.float32),
        ],
        compiler_params=pltpu.CompilerParams(
            dimension_semantics=("arbitrary",),
        ),
    )(ids3, feat, W_gate, W_feat, bf2)


# speedup vs baseline: 19.6172x; 2.9400x over previous
import jax
import jax.numpy as jnp
from jax.experimental import pallas as pl
from jax.experimental.pallas import tpu as pltpu

_N = 100000
_D = 128
_H = 128
_B = 64
_R = 4000
_NBLK = _N // _R

_HI = jax.lax.Precision.DEFAULT


def _body(ids_ref, feat_ref, wg_ref, wf_ref, bf_ref, out_ref,
          m_ref, s_ref, v_ref):
    i = pl.program_id(0)
    nb = pl.num_programs(0)

    @pl.when(i == 0)
    def _init():
        m_ref[...] = jnp.full_like(m_ref, -jnp.inf)
        s_ref[...] = jnp.zeros_like(s_ref)
        v_ref[...] = jnp.zeros_like(v_ref)

    feat = feat_ref[...]
    ids = ids_ref[0, 0, :]

    g = jax.lax.dot_general(wg_ref[...], feat, (((0,), (1,)), ((), ())),
                            preferred_element_type=jnp.float32,
                            precision=_HI)

    onehot_b = jax.lax.broadcasted_iota(jnp.int32, (_B, _R), 0) == ids[None, :]
    onehot = onehot_b.astype(jnp.float32)

    m_old = m_ref[...]
    m_blk = jnp.max(jnp.where(onehot_b, g, -jnp.inf), axis=1, keepdims=True)
    m_new = jnp.maximum(m_old, m_blk)
    scale = jnp.where(m_old == -jnp.inf, 0.0, jnp.exp(m_old - m_new))

    m_safe = jnp.where(m_new == -jnp.inf, 0.0, m_new)
    m_gat = jax.lax.dot_general(m_safe, onehot, (((0,), (0,)), ((), ())),
                                preferred_element_type=jnp.float32,
                                precision=_HI)
    e = jnp.exp(g - m_gat)
    we = onehot * e

    s_ref[...] = s_ref[...] * scale + jnp.sum(we, axis=1, keepdims=True)
    v_blk = jax.lax.dot_general(we, feat, (((1,), (0,)), ((), ())),
                                preferred_element_type=jnp.float32,
                                precision=_HI)
    v_ref[...] = v_ref[...] * scale + v_blk
    m_ref[...] = m_new

    @pl.when(i == nb - 1)
    def _finish():
        s = s_ref[...]
        pooled = jnp.where(s > 0, v_ref[...] / jnp.where(s > 0, s, 1.0), 0.0)
        out_ref[...] = jax.lax.dot_general(
            pooled, wf_ref[...], (((1,), (0,)), ((), ())),
            preferred_element_type=jnp.float32, precision=_HI) + bf_ref[...]


def kernel(feat, segment_ids, W_gate, W_feat, b_feat):
    ids3 = segment_ids.reshape(_NBLK, 1, _R)
    bf2 = b_feat.reshape(1, _H)
    return pl.pallas_call(
        _body,
        grid=(_NBLK,),
        in_specs=[
            pl.BlockSpec((1, 1, _R), lambda i: (i, 0, 0)),
            pl.BlockSpec((_R, _D), lambda i: (i, 0)),
            pl.BlockSpec((_D, 1), lambda i: (0, 0)),
            pl.BlockSpec((_D, _H), lambda i: (0, 0)),
            pl.BlockSpec((1, _H), lambda i: (0, 0)),
        ],
        out_specs=pl.BlockSpec((_B, _H), lambda i: (0, 0)),
        out_shape=jax.ShapeDtypeStruct((_B, _H), jnp.float32),
        scratch_shapes=[
            pltpu.VMEM((_B, 1), jnp.float32),
            pltpu.VMEM((_B, 1), jnp.float32),
            pltpu.VMEM((_B, _H), jnp.float32),
        ],
        compiler_params=pltpu.CompilerParams(
            dimension_semantics=("arbitrary",),
        ),
    )(ids3, feat, W_gate, W_feat, bf2)


# speedup vs baseline: 26.3888x; 1.3452x over previous
import jax
import jax.numpy as jnp
from jax.experimental import pallas as pl
from jax.experimental.pallas import tpu as pltpu

_N = 100000
_D = 128
_H = 128
_B = 64
_R = 10000
_NBLK = _N // _R

_HI = jax.lax.Precision.DEFAULT


def _body(ids_ref, feat_ref, wg_ref, wf_ref, bf_ref, out_ref,
          m_ref, s_ref, v_ref):
    i = pl.program_id(0)
    nb = pl.num_programs(0)

    @pl.when(i == 0)
    def _init():
        m_ref[...] = jnp.full_like(m_ref, -jnp.inf)
        s_ref[...] = jnp.zeros_like(s_ref)
        v_ref[...] = jnp.zeros_like(v_ref)

    feat = feat_ref[...]
    ids = ids_ref[0, 0, :]

    g = jax.lax.dot_general(wg_ref[...], feat, (((0,), (1,)), ((), ())),
                            preferred_element_type=jnp.float32,
                            precision=_HI)

    onehot_b = jax.lax.broadcasted_iota(jnp.int32, (_B, _R), 0) == ids[None, :]
    onehot = onehot_b.astype(jnp.float32)

    m_old = m_ref[...]
    m_blk = jnp.max(jnp.where(onehot_b, g, -jnp.inf), axis=1, keepdims=True)
    m_new = jnp.maximum(m_old, m_blk)
    scale = jnp.where(m_old == -jnp.inf, 0.0, jnp.exp(m_old - m_new))

    m_safe = jnp.where(m_new == -jnp.inf, 0.0, m_new)
    m_gat = jax.lax.dot_general(m_safe, onehot, (((0,), (0,)), ((), ())),
                                preferred_element_type=jnp.float32,
                                precision=_HI)
    e = jnp.exp(g - m_gat)
    we = onehot * e

    s_ref[...] = s_ref[...] * scale + jnp.sum(we, axis=1, keepdims=True)
    v_blk = jax.lax.dot_general(we, feat, (((1,), (0,)), ((), ())),
                                preferred_element_type=jnp.float32,
                                precision=_HI)
    v_ref[...] = v_ref[...] * scale + v_blk
    m_ref[...] = m_new

    @pl.when(i == nb - 1)
    def _finish():
        s = s_ref[...]
        pooled = jnp.where(s > 0, v_ref[...] / jnp.where(s > 0, s, 1.0), 0.0)
        out_ref[...] = jax.lax.dot_general(
            pooled, wf_ref[...], (((1,), (0,)), ((), ())),
            preferred_element_type=jnp.float32, precision=_HI) + bf_ref[...]


def kernel(feat, segment_ids, W_gate, W_feat, b_feat):
    ids3 = segment_ids.reshape(_NBLK, 1, _R)
    bf2 = b_feat.reshape(1, _H)
    return pl.pallas_call(
        _body,
        grid=(_NBLK,),
        in_specs=[
            pl.BlockSpec((1, 1, _R), lambda i: (i, 0, 0)),
            pl.BlockSpec((_R, _D), lambda i: (i, 0)),
            pl.BlockSpec((_D, 1), lambda i: (0, 0)),
            pl.BlockSpec((_D, _H), lambda i: (0, 0)),
            pl.BlockSpec((1, _H), lambda i: (0, 0)),
        ],
        out_specs=pl.BlockSpec((_B, _H), lambda i: (0, 0)),
        out_shape=jax.ShapeDtypeStruct((_B, _H), jnp.float32),
        scratch_shapes=[
            pltpu.VMEM((_B, 1), jnp.float32),
            pltpu.VMEM((_B, 1), jnp.float32),
            pltpu.VMEM((_B, _H), jnp.float32),
        ],
        compiler_params=pltpu.CompilerParams(
            dimension_semantics=("arbitrary",),
        ),
    )(ids3, feat, W_gate, W_feat, bf2)
